# Initial kernel scaffold; baseline (speedup 1.0000x reference)
import jax
import jax.numpy as jnp
from jax.experimental import pallas as pl


def kernel(x, edge_index, edge_weight, W, b):
    raise NotImplementedError("write your pallas kernel here")


# speedup vs baseline: 5.2633x; 5.2633x over previous
import functools

import jax
import jax.numpy as jnp
from jax import lax
from jax.experimental import pallas as pl
from jax.experimental.pallas import tpu as pltpu
from jax.experimental.pallas import tpu_sc as plsc

N = 10000
E = 160000
D_IN = 256
D_OUT = 256
H = 128
NT = 16
EPT = E // NT
CH = 16
SUP = 400
NSUP = EPT // SUP
SUPCH = SUP // CH
DEG_CH = 80
SUPDEG = SUP // DEG_CH
NRC = N // 16
RR = NRC // NT + 1


def _rsqrt16(d):
    i = lax.bitcast_convert_type(d, jnp.int32)
    i = jnp.int32(0x5F3759DF) - lax.shift_right_logical(i, 1)
    y = lax.bitcast_convert_type(i, jnp.float32)
    for _ in range(3):
        y = y * (1.5 - 0.5 * d * y * y)
    return y


def _mm_body(x_ref, w_ref, o_ref):
    o_ref[0] = jnp.dot(x_ref[...], w_ref[...], preferred_element_type=jnp.float32)


def _xlin_split(x, W):
    BN = 400
    out = pl.pallas_call(
        _mm_body,
        grid=(2, N // BN),
        in_specs=[
            pl.BlockSpec((BN, D_IN), lambda h, i: (i, 0)),
            pl.BlockSpec((D_IN, H), lambda h, i: (0, h)),
        ],
        out_specs=pl.BlockSpec((1, BN, H), lambda h, i: (h, i, 0)),
        out_shape=jax.ShapeDtypeStruct((2, N, H), jnp.float32),
    )(x, W)
    return out.reshape(2 * N, H)


_mesh = plsc.VectorSubcoreMesh(core_axis_name="c", subcore_axis_name="s")


@functools.partial(
    pl.kernel,
    out_type=jax.ShapeDtypeStruct((2 * N, H), jnp.float32),
    mesh=_mesh,
    compiler_params=pltpu.CompilerParams(
        needs_layout_passes=False,
        use_tc_tiling_on_sc=False,
    ),
    scratch_types=[
        pltpu.VMEM_SHARED((N, H), jnp.float32),
        pltpu.VMEM_SHARED((N,), jnp.float32),
        pltpu.VMEM((N,), jnp.float32),
        pltpu.VMEM((SUPCH, CH), jnp.int32),
        pltpu.VMEM((SUPCH, CH), jnp.int32),
        pltpu.VMEM((SUPCH, CH), jnp.float32),
        pltpu.VMEM((16, H), jnp.float32),
        pltpu.VMEM((16,), jnp.float32),
        pltpu.VMEM((H,), jnp.float32),
    ],
)
def _gcn_sc(xlin, row3, col3, ew3, zeros_n, bvec, out,
            acc_spm, dg_spm, dis_v, row_b, col_b, ew_b,
            gbuf, dtmp, b_vm):
    c_idx = lax.axis_index("c")
    s_idx = lax.axis_index("s")
    half_base = c_idx * N

    pltpu.sync_copy(bvec.at[pl.ds(c_idx * H, H)], b_vm)

    @pl.when(s_idx == 0)
    def _():
        pltpu.sync_copy(zeros_n, dg_spm)

    plsc.subcore_barrier()

    def deg_sup(j, carry):
        pltpu.sync_copy(col3.at[s_idx, j], col_b)
        pltpu.sync_copy(ew3.at[s_idx, j], ew_b)
        for u in range(SUPCH):
            pltpu.sync_copy(ew_b.at[u], dg_spm.at[col_b.at[u]], add=True)
        return carry

    lax.fori_loop(0, NSUP, deg_sup, 0)
    plsc.subcore_barrier()

    def dis_chunk(i, carry):
        k = s_idx + i * NT

        @pl.when(k < NRC)
        def _():
            st = pl.multiple_of(k * 16, 16)
            pltpu.sync_copy(dg_spm.at[pl.ds(st, 16)], dtmp)
            dtmp[...] = _rsqrt16(dtmp[...] + 1.0)
            pltpu.sync_copy(dtmp, dg_spm.at[pl.ds(st, 16)])

        return carry

    lax.fori_loop(0, RR, dis_chunk, 0)
    plsc.subcore_barrier()

    pltpu.sync_copy(dg_spm, dis_v)

    def init_chunk(i, carry):
        k = s_idx + i * NT

        @pl.when(k < NRC)
        def _():
            st = pl.multiple_of(k * 16, 16)
            pltpu.sync_copy(xlin.at[pl.ds(half_base + st, 16)], gbuf)
            dv = dis_v[pl.ds(st, 16)]
            for e in range(16):
                dse = dv[e]
                s2 = dse * dse
                for q in range(H // 16):
                    sl = pl.ds(q * 16, 16)
                    gbuf[e, sl] = gbuf[e, sl] * s2
            pltpu.sync_copy(gbuf, acc_spm.at[pl.ds(st, 16)])

        return carry

    lax.fori_loop(0, RR, init_chunk, 0)
    plsc.subcore_barrier()

    def edge_sup(j, carry):
        pltpu.sync_copy(row3.at[s_idx, j], row_b)
        pltpu.sync_copy(col3.at[s_idx, j], col_b)
        pltpu.sync_copy(ew3.at[s_idx, j], ew_b)

        def edge_chunk(u, carry2):
            row16 = row_b[u]
            col16 = col_b[u]
            ew16 = ew_b[u]
            pltpu.sync_copy(xlin.at[row16 + half_base], gbuf)
            dr = plsc.load_gather(dis_v, [row16])
            dc = plsc.load_gather(dis_v, [col16])
            norm16 = dr * ew16 * dc
            for e in range(CH):
                ne = norm16[e]
                for q in range(H // 16):
                    sl = pl.ds(q * 16, 16)
                    gbuf[e, sl] = gbuf[e, sl] * ne
            pltpu.sync_copy(gbuf, acc_spm.at[col_b.at[u]], add=True)
            return carry2

        lax.fori_loop(0, SUPCH, edge_chunk, 0)
        return carry

    lax.fori_loop(0, NSUP, edge_sup, 0)
    plsc.subcore_barrier()

    def out_chunk(i, carry):
        k = s_idx + i * NT

        @pl.when(k < NRC)
        def _():
            st = pl.multiple_of(k * 16, 16)
            pltpu.sync_copy(acc_spm.at[pl.ds(st, 16)], gbuf)
            for e in range(16):
                for q in range(H // 16):
                    sl = pl.ds(q * 16, 16)
                    gbuf[e, sl] = jnp.maximum(gbuf[e, sl] + b_vm[sl], 0.0)
            pltpu.sync_copy(gbuf, out.at[pl.ds(half_base + st, 16)])

        return carry

    lax.fori_loop(0, RR, out_chunk, 0)


def kernel(x, edge_index, edge_weight, W, b):
    x = x.astype(jnp.float32)
    W = W.astype(jnp.float32)
    ew = edge_weight.astype(jnp.float32)
    b = b.astype(jnp.float32)
    row = edge_index[0].astype(jnp.int32)
    col = edge_index[1].astype(jnp.int32)

    xlin = _xlin_split(x, W)
    row3 = row.reshape(NT, NSUP, SUPCH, CH)
    col3 = col.reshape(NT, NSUP, SUPCH, CH)
    ew3 = ew.reshape(NT, NSUP, SUPCH, CH)
    zeros_n = jnp.zeros((N,), jnp.float32)

    out2 = _gcn_sc(xlin, row3, col3, ew3, zeros_n, b)
    return out2.reshape(2, N, H).transpose(1, 0, 2).reshape(N, D_OUT)


# speedup vs baseline: 6.9229x; 1.3153x over previous
import functools

import jax
import jax.numpy as jnp
from jax import lax
from jax.experimental import pallas as pl
from jax.experimental.pallas import tpu as pltpu
from jax.experimental.pallas import tpu_sc as plsc

N = 10000
E = 160000
D_IN = 256
D_OUT = 256
H = 128
NT = 16
EPT = E // NT
CH = 16
NCH = EPT // CH
NPAIR = (NCH - 1) // 2
NRC = N // 16
RR = NRC // NT + 1


def _rsqrt16(d):
    i = lax.bitcast_convert_type(d, jnp.int32)
    i = jnp.int32(0x5F3759DF) - lax.shift_right_logical(i, 1)
    y = lax.bitcast_convert_type(i, jnp.float32)
    for _ in range(3):
        y = y * (1.5 - 0.5 * d * y * y)
    return y


def _mm_body(x_ref, w_ref, o_ref):
    o_ref[0] = jnp.dot(x_ref[...], w_ref[...], preferred_element_type=jnp.float32)


def _xlin_split(x, W):
    BN = 400
    out = pl.pallas_call(
        _mm_body,
        grid=(2, N // BN),
        in_specs=[
            pl.BlockSpec((BN, D_IN), lambda h, i: (i, 0)),
            pl.BlockSpec((D_IN, H), lambda h, i: (0, h)),
        ],
        out_specs=pl.BlockSpec((1, BN, H), lambda h, i: (h, i, 0)),
        out_shape=jax.ShapeDtypeStruct((2, N, H), jnp.float32),
    )(x, W)
    return out.reshape(2 * N, H)


_mesh = plsc.VectorSubcoreMesh(core_axis_name="c", subcore_axis_name="s")


@functools.partial(
    pl.kernel,
    out_type=jax.ShapeDtypeStruct((2 * N, H), jnp.float32),
    mesh=_mesh,
    compiler_params=pltpu.CompilerParams(
        needs_layout_passes=False,
        use_tc_tiling_on_sc=False,
    ),
    scratch_types=[
        pltpu.VMEM_SHARED((N, H), jnp.float32),
        pltpu.VMEM_SHARED((N,), jnp.float32),
        pltpu.VMEM((N,), jnp.float32),
        pltpu.VMEM((NCH, CH), jnp.int32),
        pltpu.VMEM((NCH, CH), jnp.int32),
        pltpu.VMEM((NCH, CH), jnp.float32),
        pltpu.VMEM((CH, H), jnp.float32),
        pltpu.VMEM((CH, H), jnp.float32),
        pltpu.VMEM((16,), jnp.float32),
        pltpu.VMEM((H,), jnp.float32),
        pltpu.SemaphoreType.DMA,
        pltpu.SemaphoreType.DMA,
    ],
)
def _gcn_sc(xlin, row3, col3, ew3, zeros_n, bvec, out,
            acc_spm, dg_spm, dis_v, row_a, col_a, ew_a,
            gbuf_a, gbuf_b, dtmp, b_vm, sem_a, sem_b):
    c_idx = lax.axis_index("c")
    s_idx = lax.axis_index("s")
    half_base = c_idx * N

    pltpu.sync_copy(bvec.at[pl.ds(c_idx * H, H)], b_vm)
    pltpu.sync_copy(row3.at[s_idx], row_a)
    pltpu.sync_copy(col3.at[s_idx], col_a)
    pltpu.sync_copy(ew3.at[s_idx], ew_a)

    @pl.when(s_idx == 0)
    def _():
        pltpu.sync_copy(zeros_n, dg_spm)

    plsc.subcore_barrier()

    def deg_chunk(u, carry):
        pltpu.sync_copy(ew_a.at[u], dg_spm.at[col_a.at[u]], add=True)
        return carry

    lax.fori_loop(0, NCH, deg_chunk, 0)
    plsc.subcore_barrier()

    def dis_chunk(i, carry):
        k = s_idx + i * NT

        @pl.when(k < NRC)
        def _():
            st = pl.multiple_of(k * 16, 16)
            pltpu.sync_copy(dg_spm.at[pl.ds(st, 16)], dtmp)
            dtmp[...] = _rsqrt16(dtmp[...] + 1.0)
            pltpu.sync_copy(dtmp, dg_spm.at[pl.ds(st, 16)])

        return carry

    lax.fori_loop(0, RR, dis_chunk, 0)
    plsc.subcore_barrier()

    pltpu.sync_copy(dg_spm, dis_v)

    def prep_chunk(u, carry):
        r16 = row_a[u]
        c16 = col_a[u]
        w16 = ew_a[u]
        dr = plsc.load_gather(dis_v, [r16])
        dc = plsc.load_gather(dis_v, [c16])
        ew_a[u] = dr * w16 * dc
        row_a[u] = r16 + half_base
        return carry

    lax.fori_loop(0, NCH, prep_chunk, 0)

    def init_chunk(i, carry):
        k = s_idx + i * NT

        @pl.when(k < NRC)
        def _():
            st = pl.multiple_of(k * 16, 16)
            pltpu.sync_copy(xlin.at[pl.ds(half_base + st, 16)], gbuf_a)
            dv = dis_v[pl.ds(st, 16)]
            for e in range(16):
                dse = dv[e]
                s2 = dse * dse
                for q in range(H // 16):
                    sl = pl.ds(q * 16, 16)
                    gbuf_a[e, sl] = gbuf_a[e, sl] * s2
            pltpu.sync_copy(gbuf_a, acc_spm.at[pl.ds(st, 16)])

        return carry

    lax.fori_loop(0, RR, init_chunk, 0)
    plsc.subcore_barrier()

    def issue(u, buf, sem):
        return pltpu.async_copy(xlin.at[row_a[u]], buf, sem)

    def wait(u, buf, sem):
        pltpu.make_async_copy(xlin.at[row_a[u]], buf, sem).wait()

    def process(u, buf):
        norm16 = ew_a[u]
        for e in range(CH):
            ne = norm16[e]
            for q in range(H // 16):
                sl = pl.ds(q * 16, 16)
                buf[e, sl] = buf[e, sl] * ne
        pltpu.sync_copy(buf, acc_spm.at[col_a.at[u]], add=True)

    issue(0, gbuf_a, sem_a)

    def pair_body(p, carry):
        u0 = p * 2
        wait(u0, gbuf_a, sem_a)
        issue(u0 + 1, gbuf_b, sem_b)
        process(u0, gbuf_a)
        wait(u0 + 1, gbuf_b, sem_b)
        issue(u0 + 2, gbuf_a, sem_a)
        process(u0 + 1, gbuf_b)
        return carry

    lax.fori_loop(0, NPAIR, pair_body, 0)
    wait(NCH - 1, gbuf_a, sem_a)
    process(NCH - 1, gbuf_a)
    plsc.subcore_barrier()

    def out_chunk(i, carry):
        k = s_idx + i * NT

        @pl.when(k < NRC)
        def _():
            st = pl.multiple_of(k * 16, 16)
            pltpu.sync_copy(acc_spm.at[pl.ds(st, 16)], gbuf_a)
            for e in range(16):
                for q in range(H // 16):
                    sl = pl.ds(q * 16, 16)
                    gbuf_a[e, sl] = jnp.maximum(gbuf_a[e, sl] + b_vm[sl], 0.0)
            pltpu.sync_copy(gbuf_a, out.at[pl.ds(half_base + st, 16)])

        return carry

    lax.fori_loop(0, RR, out_chunk, 0)


def kernel(x, edge_index, edge_weight, W, b):
    x = x.astype(jnp.float32)
    W = W.astype(jnp.float32)
    ew = edge_weight.astype(jnp.float32)
    b = b.astype(jnp.float32)
    row = edge_index[0].astype(jnp.int32)
    col = edge_index[1].astype(jnp.int32)

    xlin = _xlin_split(x, W)
    row3 = row.reshape(NT, NCH, CH)
    col3 = col.reshape(NT, NCH, CH)
    ew3 = ew.reshape(NT, NCH, CH)
    zeros_n = jnp.zeros((N,), jnp.float32)

    out2 = _gcn_sc(xlin, row3, col3, ew3, zeros_n, b)
    return out2.reshape(2, N, H).transpose(1, 0, 2).reshape(N, D_OUT)
